# Initial kernel scaffold; baseline (speedup 1.0000x reference)
import jax
import jax.numpy as jnp
from jax.experimental import pallas as pl


def kernel(nodes, edges, globals_, senders, receivers, W_e1, b_e1, W_e2, b_e2, W_n1, b_n1, W_n2, b_n2):
    raise NotImplementedError("write your pallas kernel here")


# speedup vs baseline: 2.0556x; 2.0556x over previous
import functools

import jax
import jax.numpy as jnp
from jax import lax
from jax.experimental import pallas as pl
from jax.experimental.pallas import tpu as pltpu
from jax.experimental.pallas import tpu_sc as plsc

NC = 2
NS = 16
NW = NC * NS

_mesh = lambda: plsc.VectorSubcoreMesh(core_axis_name="c", subcore_axis_name="s")


def _sc_gather(nodes, senders, receivers):
    n, d = nodes.shape
    e = senders.shape[0]
    per_w = e // NW
    ch = 40
    n_it = per_w // ch

    @functools.partial(
        pl.kernel,
        mesh=_mesh(),
        out_type=jax.ShapeDtypeStruct((e, 2 * d), jnp.float32),
        scratch_types=[
            pltpu.VMEM((ch,), jnp.int32),
            pltpu.VMEM((ch,), jnp.int32),
            pltpu.VMEM((ch, d), jnp.float32),
            pltpu.VMEM((ch, d), jnp.float32),
            pltpu.SemaphoreType.DMA,
            pltpu.SemaphoreType.DMA,
        ],
    )
    def k(nodes_hbm, s_hbm, r_hbm, g_hbm, sidx, ridx, sbuf, rbuf, sem_s, sem_r):
        wid = lax.axis_index("s") * NC + lax.axis_index("c")
        base = wid * per_w

        def body(i, carry):
            e0 = base + i * ch
            pltpu.sync_copy(s_hbm.at[pl.ds(e0, ch)], sidx)
            pltpu.sync_copy(r_hbm.at[pl.ds(e0, ch)], ridx)
            cs = pltpu.async_copy(nodes_hbm.at[sidx], sbuf, sem_s)
            cr = pltpu.async_copy(nodes_hbm.at[ridx], rbuf, sem_r)
            cs.wait()
            cr.wait()
            pltpu.sync_copy(sbuf, g_hbm.at[pl.ds(e0, ch), pl.ds(0, d)])
            pltpu.sync_copy(rbuf, g_hbm.at[pl.ds(e0, ch), pl.ds(d, d)])
            return carry

        lax.fori_loop(0, n_it, body, 0)

    return k(nodes, senders, receivers)


def _sc_segsum(vals, idx, n_seg):
    e, f = vals.shape
    fb = f // NC
    per_t = e // NS
    ch = 200
    n_it = per_t // ch
    n_pad = ((n_seg + NS * 8 - 1) // (NS * 8)) * (NS * 8)
    rows_t = n_pad // NS
    zeros = jnp.zeros((rows_t, fb), jnp.float32)

    @functools.partial(
        pl.kernel,
        mesh=_mesh(),
        out_type=jax.ShapeDtypeStruct((n_pad, f), jnp.float32),
        scratch_types=[
            pltpu.VMEM((ch,), jnp.int32),
            pltpu.VMEM((ch, fb), jnp.float32),
            pltpu.VMEM_SHARED((n_pad, fb), jnp.float32),
        ],
    )
    def k(v_hbm, i_hbm, z_hbm, out_hbm, idxbuf, rowsbuf, table):
        c = lax.axis_index("c")
        sid = lax.axis_index("s")
        r0 = sid * rows_t
        pltpu.sync_copy(z_hbm, table.at[pl.ds(r0, rows_t)])
        plsc.subcore_barrier()

        base = sid * per_t

        def body(i, carry):
            e0 = base + i * ch
            pltpu.sync_copy(i_hbm.at[pl.ds(e0, ch)], idxbuf)
            pltpu.sync_copy(v_hbm.at[pl.ds(e0, ch), pl.ds(c * fb, fb)], rowsbuf)
            pltpu.sync_copy(rowsbuf, table.at[idxbuf], add=True)
            return carry

        lax.fori_loop(0, n_it, body, 0)
        plsc.subcore_barrier()
        pltpu.sync_copy(table.at[pl.ds(r0, rows_t)],
                        out_hbm.at[pl.ds(r0, rows_t), pl.ds(c * fb, fb)])

    return k(vals, idx, zeros)[:n_seg]


def _edge_body(e_ref, g_ref, w1e_ref, w1sr_ref, w1g_ref, gl_ref, b1_ref,
               w2_ref, b2_ref, o_ref):
    acc = jnp.dot(e_ref[...], w1e_ref[...], preferred_element_type=jnp.float32)
    acc += jnp.dot(g_ref[...], w1sr_ref[...], preferred_element_type=jnp.float32)
    acc += jnp.dot(gl_ref[...], w1g_ref[...], preferred_element_type=jnp.float32)
    h = jnp.maximum(acc + b1_ref[...], 0.0)
    o_ref[...] = (jnp.dot(h, w2_ref[...], preferred_element_type=jnp.float32)
                  + b2_ref[...])


def _tc_edge(edges, g, w1e, w1sr, w1g, gl, b1, w2, b2):
    e, de = edges.shape
    dg = g.shape[1]
    h = w1e.shape[1]
    eo = w2.shape[1]
    blk = 640
    grid = e // blk
    full = lambda i: (0, 0)
    return pl.pallas_call(
        _edge_body,
        grid=(grid,),
        in_specs=[
            pl.BlockSpec((blk, de), lambda i: (i, 0)),
            pl.BlockSpec((blk, dg), lambda i: (i, 0)),
            pl.BlockSpec(w1e.shape, full),
            pl.BlockSpec(w1sr.shape, full),
            pl.BlockSpec(w1g.shape, full),
            pl.BlockSpec(gl.shape, full),
            pl.BlockSpec(b1.shape, full),
            pl.BlockSpec(w2.shape, full),
            pl.BlockSpec(b2.shape, full),
        ],
        out_specs=pl.BlockSpec((blk, eo), lambda i: (i, 0)),
        out_shape=jax.ShapeDtypeStruct((e, eo), jnp.float32),
    )(edges, g, w1e, w1sr, w1g, gl, b1, w2, b2)


def _node_body(n_ref, s_ref, r_ref, wa_ref, wb_ref, wc_ref, wg_ref, gl_ref,
               b1_ref, w2_ref, b2_ref, o_ref):
    acc = jnp.dot(n_ref[...], wa_ref[...], preferred_element_type=jnp.float32)
    acc += jnp.dot(s_ref[...], wb_ref[...], preferred_element_type=jnp.float32)
    acc += jnp.dot(r_ref[...], wc_ref[...], preferred_element_type=jnp.float32)
    acc += jnp.dot(gl_ref[...], wg_ref[...], preferred_element_type=jnp.float32)
    h = jnp.maximum(acc + b1_ref[...], 0.0)
    o_ref[...] = (jnp.dot(h, w2_ref[...], preferred_element_type=jnp.float32)
                  + b2_ref[...] + n_ref[...])


def _tc_node(nodes, agg_s, agg_r, wa, wb, wc, wg, gl, b1, w2, b2):
    n, dn = nodes.shape
    no = w2.shape[1]
    blk = 1000
    grid = n // blk
    full = lambda i: (0, 0)
    return pl.pallas_call(
        _node_body,
        grid=(grid,),
        in_specs=[
            pl.BlockSpec((blk, dn), lambda i: (i, 0)),
            pl.BlockSpec((blk, agg_s.shape[1]), lambda i: (i, 0)),
            pl.BlockSpec((blk, agg_r.shape[1]), lambda i: (i, 0)),
            pl.BlockSpec(wa.shape, full),
            pl.BlockSpec(wb.shape, full),
            pl.BlockSpec(wc.shape, full),
            pl.BlockSpec(wg.shape, full),
            pl.BlockSpec(gl.shape, full),
            pl.BlockSpec(b1.shape, full),
            pl.BlockSpec(w2.shape, full),
            pl.BlockSpec(b2.shape, full),
        ],
        out_specs=pl.BlockSpec((blk, no), lambda i: (i, 0)),
        out_shape=jax.ShapeDtypeStruct((n, no), jnp.float32),
    )(nodes, agg_s, agg_r, wa, wb, wc, wg, gl, b1, w2, b2)


def kernel(nodes, edges, globals_, senders, receivers,
           W_e1, b_e1, W_e2, b_e2, W_n1, b_n1, W_n2, b_n2):
    n, dn = nodes.shape
    de = edges.shape[1]
    dg = globals_.shape[1]
    gl = globals_.reshape(1, dg).astype(jnp.float32)

    w1e = W_e1[:de]
    w1sr = W_e1[de:de + 2 * dn]
    w1g = W_e1[de + 2 * dn:]

    g = _sc_gather(nodes, senders, receivers)
    new_edges = _tc_edge(edges, g, w1e, w1sr, w1g, gl,
                         b_e1.reshape(1, -1), W_e2, b_e2.reshape(1, -1))

    agg_s = _sc_segsum(new_edges, senders, n)
    agg_r = _sc_segsum(new_edges, receivers, n)

    eo = new_edges.shape[1]
    wa = W_n1[:dn]
    wb = W_n1[dn:dn + eo]
    wc = W_n1[dn + eo:dn + 2 * eo]
    wg = W_n1[dn + 2 * eo:]

    out_nodes = _tc_node(nodes, agg_s, agg_r, wa, wb, wc, wg, gl,
                         b_n1.reshape(1, -1), W_n2, b_n2.reshape(1, -1))
    return (out_nodes, edges, globals_)
